# baseline (speedup 1.0000x reference)
import functools

import jax
import jax.numpy as jnp
from jax import lax
from jax.experimental import pallas as pl
from jax.experimental.pallas import tpu as pltpu
from jax.experimental.pallas import tpu_sc as plsc

NC = 2
NS = 16
B = 80
R = 2


def _matmul_body(x_ref, w_ref, o_ref):
    o_ref[...] = jnp.dot(x_ref[...], w_ref[...],
                         preferred_element_type=jnp.float32)


def _matmul_bias_bf_body(x_ref, w_ref, b_ref, o_ref):
    z = (jnp.dot(x_ref[...], w_ref[...], preferred_element_type=jnp.float32)
         + b_ref[...])
    z3 = z.reshape(z.shape[0] // 2, 2, z.shape[1])
    zb0 = z3[:, 0, :].astype(jnp.bfloat16).astype(jnp.float32)
    zb1 = z3[:, 1, :].astype(jnp.bfloat16).astype(jnp.float32)
    w0 = lax.bitcast_convert_type(zb0, jnp.int32)
    w1 = lax.bitcast_convert_type(zb1, jnp.int32)
    o_ref[...] = jnp.bitwise_or(
        lax.shift_right_logical(w0, 16),
        jnp.bitwise_and(w1, jnp.int32(-65536)))


def _tc_matmul(x, w, block_rows):
    m, k = x.shape
    n = w.shape[1]
    return pl.pallas_call(
        _matmul_body,
        grid=(m // block_rows,),
        in_specs=[
            pl.BlockSpec((block_rows, k), lambda i: (i, 0)),
            pl.BlockSpec((k, n), lambda i: (0, 0)),
        ],
        out_specs=pl.BlockSpec((block_rows, n), lambda i: (i, 0)),
        out_shape=jax.ShapeDtypeStruct((m, n), jnp.float32),
    )(x, w)


def _tc_matmul_bias_bf(x, w, b, block_rows):
    m, k = x.shape
    n = w.shape[1]
    return pl.pallas_call(
        _matmul_bias_bf_body,
        grid=(m // block_rows,),
        in_specs=[
            pl.BlockSpec((block_rows, k), lambda i: (i, 0)),
            pl.BlockSpec((k, n), lambda i: (0, 0)),
            pl.BlockSpec((1, n), lambda i: (0, 0)),
        ],
        out_specs=pl.BlockSpec((block_rows // 2, n), lambda i: (i, 0)),
        out_shape=jax.ShapeDtypeStruct((m // 2, n), jnp.int32),
    )(x, w, b)


def _make_sc_edge_kernel(n_pad, d, e_pw, n_blk, rpt):
    mesh = plsc.VectorSubcoreMesh(core_axis_name="c", subcore_axis_name="s")
    assert n_blk % R == 0

    @functools.partial(
        pl.kernel,
        out_type=[
            jax.ShapeDtypeStruct((NC, n_pad, d), jnp.float32),
            jax.ShapeDtypeStruct((NC * n_pad,), jnp.float32),
        ],
        mesh=mesh,
        scratch_types=(
            [pltpu.VMEM((B,), jnp.int32)] * R
            + [pltpu.VMEM((B,), jnp.int32)] * R
            + [pltpu.VMEM((B, d), jnp.float32)] * R
            + [pltpu.VMEM((B // 2, d), jnp.int32)] * R
            + [
                pltpu.VMEM((B,), jnp.float32),
                pltpu.VMEM((-(-rpt // 16) * 16,), jnp.float32),
                pltpu.VMEM_SHARED((n_pad, d), jnp.float32),
                pltpu.VMEM_SHARED((n_pad,), jnp.float32),
            ]
            + [pltpu.SemaphoreType.DMA] * (6 * R)
        ),
    )
    def sc_kernel(xa_hbm, eb_hbm, row_hbm, col_hbm, zd_hbm,
                  h_out, c_out,
                  ir0, ir1, ic0, ic1,
                  gb0, gb1, eb0, eb1,
                  ones, vbuf, hsh, csh,
                  sir0, sir1, sic0, sic1,
                  sg0, sg1, se0, se1,
                  ssh0, ssh1, ssc0, ssc1):
        IR = [ir0, ir1]
        IC = [ic0, ic1]
        GB = [gb0, gb1]
        EB = [eb0, eb1]
        SIR = [sir0, sir1]
        SIC = [sic0, sic1]
        SG = [sg0, sg1]
        SE = [se0, se1]
        SSH = [ssh0, ssh1]
        SSC = [ssc0, ssc1]

        cid = lax.axis_index("c")
        sid = lax.axis_index("s")
        wid = cid * NS + sid
        r0 = sid * rpt
        base0 = wid * e_pw

        pltpu.sync_copy(zd_hbm.at[pl.ds(r0, rpt)], hsh.at[pl.ds(r0, rpt)])

        def fill_ones(i, carry):
            ones[pl.ds(i * 16, 16)] = jnp.full((16,), 1.0, dtype=jnp.float32)
            return carry
        lax.fori_loop(0, B // 16, fill_ones, 0)

        def fill_zero(i, carry):
            vbuf[pl.ds(i * 16, 16)] = jnp.zeros((16,), dtype=jnp.float32)
            return carry
        lax.fori_loop(0, vbuf.shape[0] // 16, fill_zero, 0)
        pltpu.sync_copy(vbuf.at[pl.ds(0, rpt)], csh.at[pl.ds(r0, rpt)])

        plsc.subcore_barrier()

        def issue_loads(j, s):
            base = base0 + j * B
            pltpu.async_copy(row_hbm.at[pl.ds(base, B)], IR[s], SIR[s])
            pltpu.async_copy(col_hbm.at[pl.ds(base, B)], IC[s], SIC[s])
            base2 = pl.multiple_of(base // 2, 8)
            pltpu.async_copy(eb_hbm.at[pl.ds(base2, B // 2)],
                             EB[s], SE[s])

        def start_gather(s):
            pltpu.make_async_copy(
                row_hbm.at[pl.ds(0, B)], IR[s], SIR[s]).wait()
            pltpu.async_copy(xa_hbm.at[IR[s]], GB[s], SG[s])

        def drain_scatters(s):
            pltpu.make_async_copy(GB[s], hsh.at[IC[s]], SSH[s]).wait()
            pltpu.make_async_copy(ones, csh.at[IC[s]], SSC[s]).wait()

        def process(s):
            pltpu.make_async_copy(xa_hbm.at[IR[s]], GB[s], SG[s]).wait()
            pltpu.make_async_copy(
                eb_hbm.at[pl.ds(0, B // 2)], EB[s], SE[s]).wait()

            def pair(i2, c2):
                i = i2 * 2
                for k in range(d // 16):
                    o = k * 16
                    v = EB[s][i2, pl.ds(o, 16)]
                    lo = lax.bitcast_convert_type(
                        jnp.left_shift(v, 16), jnp.float32)
                    hi = lax.bitcast_convert_type(
                        jnp.bitwise_and(v, jnp.int32(-65536)),
                        jnp.float32)
                    a = GB[s][i, pl.ds(o, 16)] + lo
                    GB[s][i, pl.ds(o, 16)] = jnp.maximum(a, 0.0)
                    b = GB[s][i + 1, pl.ds(o, 16)] + hi
                    GB[s][i + 1, pl.ds(o, 16)] = jnp.maximum(b, 0.0)
                return c2
            lax.fori_loop(0, B // 2, pair, 0)

            pltpu.make_async_copy(
                col_hbm.at[pl.ds(0, B)], IC[s], SIC[s]).wait()
            pltpu.async_copy(GB[s], hsh.at[IC[s]], SSH[s], add=True)
            pltpu.async_copy(ones, csh.at[IC[s]], SSC[s], add=True)

        for s in range(R - 1):
            issue_loads(s, s)
            start_gather(s)

        def outer(k, carry):
            for p in range(R):
                j = k * R + p
                sn = (p + R - 1) % R
                process(p)

                @pl.when(j + R - 1 < n_blk)
                def _prefetch():
                    @pl.when(j >= 1)
                    def _drain():
                        drain_scatters(sn)
                    issue_loads(j + R - 1, sn)
                    start_gather(sn)
            return carry
        lax.fori_loop(0, n_blk // R, outer, 0)

        for s in range(R):
            drain_scatters(s)

        plsc.subcore_barrier()

        pltpu.sync_copy(hsh.at[pl.ds(r0, rpt)], h_out.at[cid, pl.ds(r0, rpt)])
        pltpu.sync_copy(csh.at[pl.ds(r0, rpt)], vbuf.at[pl.ds(0, rpt)])
        pltpu.sync_copy(vbuf.at[pl.ds(0, rpt)],
                        c_out.at[pl.ds(cid * n_pad + r0, rpt)])

    return sc_kernel


def _final_body(x_ref, h0_ref, h1_ref, c0_ref, c1_ref, b_ref, u_ref,
                w2_ref, b2_ref, w3a_ref, w3b_ref, w3c_ref, b3_ref,
                w4_ref, b4_ref, o_ref):
    cnt = (c0_ref[0, 0, :] + c1_ref[0, 0, :])[:, None]
    h = h0_ref[...] + h1_ref[...]
    hm = h / jnp.maximum(cnt, 1.0)
    gate = jnp.where(cnt > 0.5, 1.0, 0.0)
    mean = (jnp.dot(hm, w2_ref[...], preferred_element_type=jnp.float32)
            + gate * b2_ref[...])
    bvec = b_ref[0, 0, :]
    g = u_ref.shape[0]
    oneh = (bvec[:, None] == lax.broadcasted_iota(jnp.int32, (1, g), 1)
            ).astype(jnp.float32)
    ug = jnp.dot(oneh, u_ref[...], preferred_element_type=jnp.float32)
    z = (jnp.dot(x_ref[...], w3a_ref[...], preferred_element_type=jnp.float32)
         + jnp.dot(mean, w3b_ref[...], preferred_element_type=jnp.float32)
         + jnp.dot(ug, w3c_ref[...], preferred_element_type=jnp.float32)
         + b3_ref[...])
    y = jnp.dot(jnp.maximum(z, 0.0), w4_ref[...],
                preferred_element_type=jnp.float32) + b4_ref[...]
    o_ref[...] = y


def kernel(x, edge_index, edge_attr, u, batch, W1, b1, W2, b2, W3, b3, W4, b4):
    n, d_node = x.shape
    e = edge_index.shape[1]
    d_edge = edge_attr.shape[1]
    g, d_glob = u.shape
    d_out = W2.shape[0]

    nw = NC * NS
    n_blk = -(-(-(-e // (nw * B))) // R) * R
    e_pw = n_blk * B
    e_pad = e_pw * nw
    n_pad = -(-(n + 1) // (NS * 8)) * (NS * 8)
    rpt = n_pad // NS

    row = edge_index[0].astype(jnp.int32)
    col = edge_index[1].astype(jnp.int32)
    pad = e_pad - e
    row_p = jnp.concatenate([row, jnp.zeros((pad,), jnp.int32)])
    col_p = jnp.concatenate([col, jnp.full((pad,), n, jnp.int32)])
    ea_p = jnp.concatenate(
        [edge_attr, jnp.zeros((pad, d_edge), jnp.float32)])

    W1a = W1[:d_node]
    W1b = W1[d_node:]
    W3a = W3[:d_node]
    W3b = W3[d_node:d_node + d_out]
    W3c = W3[d_node + d_out:]
    zd = jnp.zeros((n_pad, d_out), jnp.float32)

    xa = _tc_matmul(x, W1a, 1000)
    eb_i32 = _tc_matmul_bias_bf(ea_p, W1b, b1.reshape(1, -1), 512)

    sc_fn = _make_sc_edge_kernel(n_pad, d_out, e_pw, n_blk, rpt)
    h_parts, c_parts = sc_fn(xa, eb_i32, row_p, col_p, zd)

    h0 = h_parts[0, :n]
    h1 = h_parts[1, :n]
    nb = n // 1000
    c0 = c_parts[:n].reshape(nb, 1, n // nb)
    c1 = c_parts[n_pad:n_pad + n].reshape(nb, 1, n // nb)
    batch3 = batch.astype(jnp.int32).reshape(nb, 1, n // nb)

    br = 1000
    y = pl.pallas_call(
        _final_body,
        grid=(nb,),
        in_specs=[
            pl.BlockSpec((br, d_node), lambda i: (i, 0)),
            pl.BlockSpec((br, d_out), lambda i: (i, 0)),
            pl.BlockSpec((br, d_out), lambda i: (i, 0)),
            pl.BlockSpec((1, 1, br), lambda i: (i, 0, 0)),
            pl.BlockSpec((1, 1, br), lambda i: (i, 0, 0)),
            pl.BlockSpec((1, 1, br), lambda i: (i, 0, 0)),
            pl.BlockSpec((g, d_glob), lambda i: (0, 0)),
            pl.BlockSpec((d_out, d_out), lambda i: (0, 0)),
            pl.BlockSpec((1, d_out), lambda i: (0, 0)),
            pl.BlockSpec((d_node, d_out), lambda i: (0, 0)),
            pl.BlockSpec((d_out, d_out), lambda i: (0, 0)),
            pl.BlockSpec((d_glob, d_out), lambda i: (0, 0)),
            pl.BlockSpec((1, d_out), lambda i: (0, 0)),
            pl.BlockSpec((d_out, d_out), lambda i: (0, 0)),
            pl.BlockSpec((1, d_out), lambda i: (0, 0)),
        ],
        out_specs=pl.BlockSpec((br, d_out), lambda i: (i, 0)),
        out_shape=jax.ShapeDtypeStruct((n, d_out), jnp.float32),
    )(x, h0, h1, c0, c1, batch3, u, W2, b2.reshape(1, -1),
      W3a, W3b, W3c, b3.reshape(1, -1), W4, b4.reshape(1, -1))
    return y


# speedup vs baseline: 1.1196x; 1.1196x over previous
import functools

import jax
import jax.numpy as jnp
from jax import lax
from jax.experimental import pallas as pl
from jax.experimental.pallas import tpu as pltpu
from jax.experimental.pallas import tpu_sc as plsc

NC = 2
NS = 16
B = 64
R = 2


def _matmul_body(x_ref, w_ref, o_ref):
    o_ref[...] = jnp.dot(x_ref[...], w_ref[...],
                         preferred_element_type=jnp.float32)


def _matmul_bias_body(x_ref, w_ref, b_ref, o_ref):
    o_ref[...] = jnp.dot(x_ref[...], w_ref[...],
                         preferred_element_type=jnp.float32) + b_ref[...]


def _tc_matmul(x, w, block_rows):
    m, k = x.shape
    n = w.shape[1]
    return pl.pallas_call(
        _matmul_body,
        grid=(m // block_rows,),
        in_specs=[
            pl.BlockSpec((block_rows, k), lambda i: (i, 0)),
            pl.BlockSpec((k, n), lambda i: (0, 0)),
        ],
        out_specs=pl.BlockSpec((block_rows, n), lambda i: (i, 0)),
        out_shape=jax.ShapeDtypeStruct((m, n), jnp.float32),
    )(x, w)


def _tc_matmul_bias(x, w, b, block_rows):
    m, k = x.shape
    n = w.shape[1]
    return pl.pallas_call(
        _matmul_bias_body,
        grid=(m // block_rows,),
        in_specs=[
            pl.BlockSpec((block_rows, k), lambda i: (i, 0)),
            pl.BlockSpec((k, n), lambda i: (0, 0)),
            pl.BlockSpec((1, n), lambda i: (0, 0)),
        ],
        out_specs=pl.BlockSpec((block_rows, n), lambda i: (i, 0)),
        out_shape=jax.ShapeDtypeStruct((m, n), jnp.float32),
    )(x, w, b)


def _make_sc_edge_kernel(n_pad, d, e_pw, n_blk, rpt):
    mesh = plsc.VectorSubcoreMesh(core_axis_name="c", subcore_axis_name="s")
    assert n_blk % R == 0

    @functools.partial(
        pl.kernel,
        out_type=[
            jax.ShapeDtypeStruct((NC, n_pad, d), jnp.float32),
            jax.ShapeDtypeStruct((NC * n_pad,), jnp.float32),
        ],
        mesh=mesh,
        scratch_types=(
            [pltpu.VMEM((B,), jnp.int32)] * R
            + [pltpu.VMEM((B,), jnp.int32)] * R
            + [pltpu.VMEM((B, d), jnp.float32)] * R
            + [pltpu.VMEM((B, d), jnp.float32)] * R
            + [
                pltpu.VMEM((B,), jnp.float32),
                pltpu.VMEM((-(-rpt // 16) * 16,), jnp.float32),
                pltpu.VMEM_SHARED((n_pad, d), jnp.float32),
                pltpu.VMEM_SHARED((n_pad,), jnp.float32),
            ]
            + [pltpu.SemaphoreType.DMA] * (6 * R)
        ),
    )
    def sc_kernel(xa_hbm, eb_hbm, row_hbm, col_hbm, zd_hbm,
                  h_out, c_out,
                  ir0, ir1, ic0, ic1,
                  gb0, gb1, eb0, eb1,
                  ones, vbuf, hsh, csh,
                  sir0, sir1, sic0, sic1,
                  sg0, sg1, se0, se1,
                  ssh0, ssh1, ssc0, ssc1):
        IR = [ir0, ir1]
        IC = [ic0, ic1]
        GB = [gb0, gb1]
        EB = [eb0, eb1]
        SIR = [sir0, sir1]
        SIC = [sic0, sic1]
        SG = [sg0, sg1]
        SE = [se0, se1]
        SSH = [ssh0, ssh1]
        SSC = [ssc0, ssc1]

        cid = lax.axis_index("c")
        sid = lax.axis_index("s")
        wid = cid * NS + sid
        r0 = sid * rpt
        base0 = wid * e_pw

        pltpu.sync_copy(zd_hbm.at[pl.ds(r0, rpt)], hsh.at[pl.ds(r0, rpt)])

        def fill_ones(i, carry):
            ones[pl.ds(i * 16, 16)] = jnp.full((16,), 1.0, dtype=jnp.float32)
            return carry
        lax.fori_loop(0, B // 16, fill_ones, 0)

        def fill_zero(i, carry):
            vbuf[pl.ds(i * 16, 16)] = jnp.zeros((16,), dtype=jnp.float32)
            return carry
        lax.fori_loop(0, vbuf.shape[0] // 16, fill_zero, 0)
        pltpu.sync_copy(vbuf.at[pl.ds(0, rpt)], csh.at[pl.ds(r0, rpt)])

        plsc.subcore_barrier()

        def issue_loads(j, s):
            base = base0 + j * B
            pltpu.async_copy(row_hbm.at[pl.ds(base, B)], IR[s], SIR[s])
            pltpu.async_copy(col_hbm.at[pl.ds(base, B)], IC[s], SIC[s])
            pltpu.async_copy(eb_hbm.at[pl.ds(base, B)], EB[s], SE[s])

        def start_gather(s):
            pltpu.make_async_copy(
                row_hbm.at[pl.ds(0, B)], IR[s], SIR[s]).wait()
            pltpu.async_copy(xa_hbm.at[IR[s]], GB[s], SG[s])

        def drain_scatters(s):
            pltpu.make_async_copy(GB[s], hsh.at[IC[s]], SSH[s]).wait()
            pltpu.make_async_copy(ones, csh.at[IC[s]], SSC[s]).wait()

        def process(s):
            pltpu.make_async_copy(xa_hbm.at[IR[s]], GB[s], SG[s]).wait()
            pltpu.make_async_copy(
                eb_hbm.at[pl.ds(0, B)], EB[s], SE[s]).wait()

            def edge(i, c2):
                for k in range(d // 16):
                    o = k * 16
                    g = GB[s][i, pl.ds(o, 16)]
                    e = EB[s][i, pl.ds(o, 16)]
                    GB[s][i, pl.ds(o, 16)] = jnp.maximum(g + e, 0.0)
                return c2
            lax.fori_loop(0, B, edge, 0)

            pltpu.make_async_copy(
                col_hbm.at[pl.ds(0, B)], IC[s], SIC[s]).wait()
            pltpu.async_copy(GB[s], hsh.at[IC[s]], SSH[s], add=True)
            pltpu.async_copy(ones, csh.at[IC[s]], SSC[s], add=True)

        for s in range(R - 1):
            issue_loads(s, s)
            start_gather(s)

        def outer(k, carry):
            for p in range(R):
                j = k * R + p
                sn = (p + R - 1) % R
                process(p)

                @pl.when(j + R - 1 < n_blk)
                def _prefetch():
                    @pl.when(j >= 1)
                    def _drain():
                        drain_scatters(sn)
                    issue_loads(j + R - 1, sn)
                    start_gather(sn)
            return carry
        lax.fori_loop(0, n_blk // R, outer, 0)

        for s in range(R):
            drain_scatters(s)

        plsc.subcore_barrier()

        pltpu.sync_copy(hsh.at[pl.ds(r0, rpt)], h_out.at[cid, pl.ds(r0, rpt)])
        pltpu.sync_copy(csh.at[pl.ds(r0, rpt)], vbuf.at[pl.ds(0, rpt)])
        pltpu.sync_copy(vbuf.at[pl.ds(0, rpt)],
                        c_out.at[pl.ds(cid * n_pad + r0, rpt)])

    return sc_kernel


def _final_body(x_ref, h_ref, c0_ref, c1_ref, b_ref, u_ref,
                w2_ref, b2_ref, w3a_ref, w3b_ref, w3c_ref, b3_ref,
                w4_ref, b4_ref, o_ref):
    cnt = (c0_ref[0, 0, :] + c1_ref[0, 0, :])[:, None]
    h = h_ref[0] + h_ref[1]
    hm = h / jnp.maximum(cnt, 1.0)
    gate = jnp.where(cnt > 0.5, 1.0, 0.0)
    mean = (jnp.dot(hm, w2_ref[...], preferred_element_type=jnp.float32)
            + gate * b2_ref[...])
    bvec = b_ref[0, 0, :]
    g = u_ref.shape[0]
    oneh = (bvec[:, None] == lax.broadcasted_iota(jnp.int32, (1, g), 1)
            ).astype(jnp.float32)
    ug = jnp.dot(oneh, u_ref[...], preferred_element_type=jnp.float32)
    z = (jnp.dot(x_ref[...], w3a_ref[...], preferred_element_type=jnp.float32)
         + jnp.dot(mean, w3b_ref[...], preferred_element_type=jnp.float32)
         + jnp.dot(ug, w3c_ref[...], preferred_element_type=jnp.float32)
         + b3_ref[...])
    y = jnp.dot(jnp.maximum(z, 0.0), w4_ref[...],
                preferred_element_type=jnp.float32) + b4_ref[...]
    o_ref[...] = y


def kernel(x, edge_index, edge_attr, u, batch, W1, b1, W2, b2, W3, b3, W4, b4):
    n, d_node = x.shape
    e = edge_index.shape[1]
    d_edge = edge_attr.shape[1]
    g, d_glob = u.shape
    d_out = W2.shape[0]

    nw = NC * NS
    n_blk = -(-(-(-e // (nw * B))) // R) * R
    e_pw = n_blk * B
    e_pad = e_pw * nw
    n_pad = -(-(n + 1) // (NS * 8)) * (NS * 8)
    rpt = n_pad // NS

    row = edge_index[0].astype(jnp.int32)
    col = edge_index[1].astype(jnp.int32)
    pad = e_pad - e
    row_p = jnp.concatenate([row, jnp.zeros((pad,), jnp.int32)])
    col_p = jnp.concatenate([col, jnp.full((pad,), n, jnp.int32)])
    ea_p = jnp.concatenate(
        [edge_attr, jnp.zeros((pad, d_edge), jnp.float32)])

    W1a = W1[:d_node]
    W1b = W1[d_node:]
    W3a = W3[:d_node]
    W3b = W3[d_node:d_node + d_out]
    W3c = W3[d_node + d_out:]
    zd = jnp.zeros((n_pad, d_out), jnp.float32)

    xa = _tc_matmul(x, W1a, 1000)
    eb = _tc_matmul_bias(ea_p, W1b, b1.reshape(1, -1), 512)

    sc_fn = _make_sc_edge_kernel(n_pad, d_out, e_pw, n_blk, rpt)
    h_parts, c_parts = sc_fn(xa, eb, row_p, col_p, zd)

    nb = n // 1000
    c0 = c_parts[:n].reshape(nb, 1, n // nb)
    c1 = c_parts[n_pad:n_pad + n].reshape(nb, 1, n // nb)
    batch3 = batch.astype(jnp.int32).reshape(nb, 1, n // nb)

    br = 1000
    y = pl.pallas_call(
        _final_body,
        grid=(nb,),
        in_specs=[
            pl.BlockSpec((br, d_node), lambda i: (i, 0)),
            pl.BlockSpec((NC, br, d_out), lambda i: (0, i, 0)),
            pl.BlockSpec((1, 1, br), lambda i: (i, 0, 0)),
            pl.BlockSpec((1, 1, br), lambda i: (i, 0, 0)),
            pl.BlockSpec((1, 1, br), lambda i: (i, 0, 0)),
            pl.BlockSpec((g, d_glob), lambda i: (0, 0)),
            pl.BlockSpec((d_out, d_out), lambda i: (0, 0)),
            pl.BlockSpec((1, d_out), lambda i: (0, 0)),
            pl.BlockSpec((d_node, d_out), lambda i: (0, 0)),
            pl.BlockSpec((d_out, d_out), lambda i: (0, 0)),
            pl.BlockSpec((d_glob, d_out), lambda i: (0, 0)),
            pl.BlockSpec((1, d_out), lambda i: (0, 0)),
            pl.BlockSpec((d_out, d_out), lambda i: (0, 0)),
            pl.BlockSpec((1, d_out), lambda i: (0, 0)),
        ],
        out_specs=pl.BlockSpec((br, d_out), lambda i: (i, 0)),
        out_shape=jax.ShapeDtypeStruct((n, d_out), jnp.float32),
    )(x, h_parts, c0, c1, batch3, u, W2, b2.reshape(1, -1),
      W3a, W3b, W3c, b3.reshape(1, -1), W4, b4.reshape(1, -1))
    return y
